# baseline (speedup 1.0000x reference)
import functools

import jax
import jax.numpy as jnp
from jax import lax
from jax.experimental import pallas as pl
from jax.experimental.pallas import tpu as pltpu
from jax.experimental.pallas import tpu_sc as plsc

_N = 10000
_D = 128
_NC = 2
_NS = 16
_NW = _NC * _NS
_K = 64
_NPAD = 10240
_RPT = _NPAD // _NS
_CHUNKS = 160
_EPAD = _NW * _CHUNKS * _K
_ROWS = _EPAD // _K


def _sc_aggregate(with_deg):
  mesh = plsc.VectorSubcoreMesh(core_axis_name="c", subcore_axis_name="s",
                                num_cores=_NC, num_subcores=_NS)
  if with_deg:
    out_type = [jax.ShapeDtypeStruct((_NC, _NPAD, _D), jnp.float32),
                jax.ShapeDtypeStruct((_NW, _NPAD), jnp.float32)]
  else:
    out_type = jax.ShapeDtypeStruct((_NC, _NPAD, _D), jnp.float32)
  scratch = [
      pltpu.VMEM((_CHUNKS, _K), jnp.int32),
      pltpu.VMEM((_CHUNKS, _K), jnp.int32),
      pltpu.VMEM((_K, _D), jnp.float32),
      pltpu.VMEM((_K, _D), jnp.float32),
      pltpu.VMEM_SHARED((_NPAD, _D), jnp.float32),
      pltpu.SemaphoreType.DMA,
      pltpu.SemaphoreType.DMA,
  ]
  if with_deg:
    scratch += [pltpu.VMEM((_NPAD,), jnp.float32)]

  def body(x_hbm, src_hbm, dst_hbm, za_hbm, *rest):
    if with_deg:
      (acc_out, deg_out, src_v, dst_v, rows0, rows1, acc_sh, sem0, sem1,
       deg_v) = rest
    else:
      (acc_out, src_v, dst_v, rows0, rows1, acc_sh, sem0, sem1) = rest
    c = lax.axis_index("c")
    s = lax.axis_index("s")
    t = c * _NS + s

    pltpu.sync_copy(src_hbm.at[pl.ds(t * _CHUNKS, _CHUNKS)], src_v)
    pltpu.sync_copy(dst_hbm.at[pl.ds(t * _CHUNKS, _CHUNKS)], dst_v)
    pltpu.sync_copy(za_hbm, acc_sh.at[pl.ds(s * _RPT, _RPT)])
    if with_deg:
      zeros16 = jnp.zeros((16,), jnp.float32)
      def zero_body(i, carry):
        deg_v[pl.ds(i * 16, 16)] = zeros16
        return carry
      lax.fori_loop(0, _NPAD // 16, zero_body, 0)
    plsc.subcore_barrier()

    def g_start(j, buf, sem):
      pltpu.async_copy(x_hbm.at[src_v.at[j]], buf, sem)

    def g_wait(buf, sem):
      pltpu.make_async_copy(x_hbm.at[src_v.at[0]], buf, sem).wait()

    ones16 = jnp.ones((16,), jnp.float32)

    def scat(j, buf):
      pltpu.sync_copy(buf, acc_sh.at[dst_v.at[j]], add=True)
      if with_deg:
        for v in range(_K // 16):
          idx = dst_v[j, pl.ds(v * 16, 16)]
          plsc.addupdate_scatter(deg_v, [idx], ones16)

    g_start(0, rows0, sem0)

    def loop_body(k, carry):
      j0 = 2 * k
      g_wait(rows0, sem0)
      g_start(j0 + 1, rows1, sem1)
      scat(j0, rows0)
      g_wait(rows1, sem1)
      g_start(lax.rem(j0 + 2, _CHUNKS), rows0, sem0)
      scat(j0 + 1, rows1)
      return carry

    lax.fori_loop(0, _CHUNKS // 2, loop_body, 0)
    g_wait(rows0, sem0)
    plsc.subcore_barrier()

    pltpu.sync_copy(acc_sh.at[pl.ds(s * _RPT, _RPT)],
                    acc_out.at[c, pl.ds(s * _RPT, _RPT)])
    if with_deg:
      pltpu.sync_copy(deg_v, deg_out.at[t])

  return pl.kernel(
      body, out_type=out_type, mesh=mesh, scratch_types=scratch,
      compiler_params=pltpu.CompilerParams(needs_layout_passes=False,
                                           use_tc_tiling_on_sc=False))


def _dense_body(relu, p_ref, dg_ref, x_ref, wl_ref, wr_ref, b_ref, o_ref):
  acc = p_ref[0] + p_ref[1]
  deg = jnp.sum(dg_ref[...], axis=0)[:, None]
  agg = acc / jnp.maximum(deg, 1.0)
  y = (jnp.dot(agg, wl_ref[...], preferred_element_type=jnp.float32)
       + jnp.dot(x_ref[...], wr_ref[...], preferred_element_type=jnp.float32)
       + b_ref[...])
  o_ref[...] = jnp.maximum(y, 0.0) if relu else y


def _tc_dense(parts, degp, xp, wlT, wrT, b, relu):
  blk = 2048
  grid = (_NPAD // blk,)
  return pl.pallas_call(
      functools.partial(_dense_body, relu),
      grid=grid,
      in_specs=[
          pl.BlockSpec((_NC, blk, _D), lambda i: (0, i, 0)),
          pl.BlockSpec((_NW, blk), lambda i: (0, i)),
          pl.BlockSpec((blk, _D), lambda i: (i, 0)),
          pl.BlockSpec((_D, _D), lambda i: (0, 0)),
          pl.BlockSpec((_D, _D), lambda i: (0, 0)),
          pl.BlockSpec((1, _D), lambda i: (0, 0)),
      ],
      out_specs=pl.BlockSpec((blk, _D), lambda i: (i, 0)),
      out_shape=jax.ShapeDtypeStruct((_NPAD, _D), jnp.float32),
  )(parts, degp, xp, wlT, wrT, b.reshape(1, _D))


def kernel(x, edge_index, node_type, W1l, b1l, W1r, W2l, b2l, W2r):
  f32 = jnp.float32
  x = x.astype(f32)
  src = edge_index[0].astype(jnp.int32)
  dst = edge_index[1].astype(jnp.int32)
  e = src.shape[0]
  pad = _EPAD - e
  pidx = jnp.arange(pad, dtype=jnp.int32)
  psrc = pidx % _N
  pdst = _N + (pidx % (_NPAD - _N))
  src2d = jnp.concatenate([src, psrc]).reshape(_ROWS, _K)
  dst2d = jnp.concatenate([dst, pdst]).reshape(_ROWS, _K)
  za = jnp.zeros((_RPT, _D), f32)
  xp = jnp.concatenate([x, jnp.zeros((_NPAD - _N, _D), f32)])

  agg = _sc_aggregate(True)
  parts1, degp = agg(xp, src2d, dst2d, za)
  h = _tc_dense(parts1, degp, xp, W1l.T, W1r.T, b1l, relu=True)
  parts2, _ = agg(h, src2d, dst2d, za)
  out = _tc_dense(parts2, degp, h, W2l.T, W2r.T, b2l, relu=False)
  return out[:_N]


# speedup vs baseline: 1.0185x; 1.0185x over previous
import functools

import jax
import jax.numpy as jnp
from jax import lax
from jax.experimental import pallas as pl
from jax.experimental.pallas import tpu as pltpu
from jax.experimental.pallas import tpu_sc as plsc

_N = 10000
_D = 128
_NC = 2
_NS = 16
_NW = _NC * _NS
_K = 64
_NPAD = 10240
_RPT = _NPAD // _NS
_CHUNKS = 158
_EPAD = _NW * _CHUNKS * _K
_ROWS = _EPAD // _K


def _sc_aggregate(with_deg):
  mesh = plsc.VectorSubcoreMesh(core_axis_name="c", subcore_axis_name="s",
                                num_cores=_NC, num_subcores=_NS)
  if with_deg:
    out_type = [jax.ShapeDtypeStruct((_NC, _NPAD, _D), jnp.float32),
                jax.ShapeDtypeStruct((_NW, _NPAD), jnp.float32)]
  else:
    out_type = jax.ShapeDtypeStruct((_NC, _NPAD, _D), jnp.float32)
  scratch = [
      pltpu.VMEM((_CHUNKS, _K), jnp.int32),
      pltpu.VMEM((_CHUNKS, _K), jnp.int32),
      pltpu.VMEM((_K, _D), jnp.float32),
      pltpu.VMEM((_K, _D), jnp.float32),
      pltpu.VMEM_SHARED((_NPAD, _D), jnp.float32),
      pltpu.SemaphoreType.DMA,
      pltpu.SemaphoreType.DMA,
  ]
  if with_deg:
    scratch += [pltpu.VMEM((_NPAD,), jnp.float32)]

  def body(x_hbm, src_hbm, dst_hbm, za_hbm, *rest):
    if with_deg:
      (acc_out, deg_out, src_v, dst_v, rows0, rows1, acc_sh, sem0, sem1,
       deg_v) = rest
    else:
      (acc_out, src_v, dst_v, rows0, rows1, acc_sh, sem0, sem1) = rest
    c = lax.axis_index("c")
    s = lax.axis_index("s")
    t = c * _NS + s

    pltpu.sync_copy(src_hbm.at[pl.ds(t * _CHUNKS, _CHUNKS)], src_v)
    pltpu.sync_copy(dst_hbm.at[pl.ds(t * _CHUNKS, _CHUNKS)], dst_v)
    pltpu.sync_copy(za_hbm, acc_sh.at[pl.ds(s * _RPT, _RPT)])
    if with_deg:
      zeros16 = jnp.zeros((16,), jnp.float32)
      def zero_body(i, carry):
        deg_v[pl.ds(i * 16, 16)] = zeros16
        return carry
      lax.fori_loop(0, _NPAD // 16, zero_body, 0)
    plsc.subcore_barrier()

    def g_start(j, buf, sem):
      pltpu.async_copy(x_hbm.at[src_v.at[j]], buf, sem)

    def g_wait(buf, sem):
      pltpu.make_async_copy(x_hbm.at[src_v.at[0]], buf, sem).wait()

    ones16 = jnp.ones((16,), jnp.float32)

    def scat(j, buf):
      pltpu.sync_copy(buf, acc_sh.at[dst_v.at[j]], add=True)
      if with_deg:
        for v in range(_K // 16):
          idx = dst_v[j, pl.ds(v * 16, 16)]
          plsc.addupdate_scatter(deg_v, [idx], ones16)

    def loop_body(k, carry):
      j0 = 2 * k
      g_start(j0, rows0, sem0)
      g_start(j0 + 1, rows1, sem1)
      g_wait(rows0, sem0)
      scat(j0, rows0)
      g_wait(rows1, sem1)
      scat(j0 + 1, rows1)
      return carry

    lax.fori_loop(0, _CHUNKS // 2, loop_body, 0)
    plsc.subcore_barrier()

    pltpu.sync_copy(acc_sh.at[pl.ds(s * _RPT, _RPT)],
                    acc_out.at[c, pl.ds(s * _RPT, _RPT)])
    if with_deg:
      pltpu.sync_copy(deg_v, deg_out.at[t])

  return pl.kernel(
      body, out_type=out_type, mesh=mesh, scratch_types=scratch,
      compiler_params=pltpu.CompilerParams(needs_layout_passes=False,
                                           use_tc_tiling_on_sc=False))


def _dense_body(relu, p_ref, dg_ref, x_ref, wl_ref, wr_ref, b_ref, o_ref):
  acc = p_ref[0] + p_ref[1]
  deg = jnp.sum(dg_ref[...], axis=0)[:, None]
  agg = acc / jnp.maximum(deg, 1.0)
  y = (jnp.dot(agg, wl_ref[...], preferred_element_type=jnp.float32)
       + jnp.dot(x_ref[...], wr_ref[...], preferred_element_type=jnp.float32)
       + b_ref[...])
  o_ref[...] = jnp.maximum(y, 0.0) if relu else y


def _tc_dense(parts, degp, xp, wlT, wrT, b, relu):
  blk = 2048
  grid = (_NPAD // blk,)
  return pl.pallas_call(
      functools.partial(_dense_body, relu),
      grid=grid,
      in_specs=[
          pl.BlockSpec((_NC, blk, _D), lambda i: (0, i, 0)),
          pl.BlockSpec((_NW, blk), lambda i: (0, i)),
          pl.BlockSpec((blk, _D), lambda i: (i, 0)),
          pl.BlockSpec((_D, _D), lambda i: (0, 0)),
          pl.BlockSpec((_D, _D), lambda i: (0, 0)),
          pl.BlockSpec((1, _D), lambda i: (0, 0)),
      ],
      out_specs=pl.BlockSpec((blk, _D), lambda i: (i, 0)),
      out_shape=jax.ShapeDtypeStruct((_NPAD, _D), jnp.float32),
  )(parts, degp, xp, wlT, wrT, b.reshape(1, _D))


def kernel(x, edge_index, node_type, W1l, b1l, W1r, W2l, b2l, W2r):
  f32 = jnp.float32
  x = x.astype(f32)
  src = edge_index[0].astype(jnp.int32)
  dst = edge_index[1].astype(jnp.int32)
  e = src.shape[0]
  pad = _EPAD - e
  pidx = jnp.arange(pad, dtype=jnp.int32)
  psrc = pidx % _N
  pdst = _N + (pidx % (_NPAD - _N))
  src2d = jnp.concatenate([src, psrc]).reshape(_ROWS, _K)
  dst2d = jnp.concatenate([dst, pdst]).reshape(_ROWS, _K)
  za = jnp.zeros((_RPT, _D), f32)
  xp = jnp.concatenate([x, jnp.zeros((_NPAD - _N, _D), f32)])

  agg = _sc_aggregate(True)
  parts1, degp = agg(xp, src2d, dst2d, za)
  h = _tc_dense(parts1, degp, xp, W1l.T, W1r.T, b1l, relu=True)
  parts2, _ = agg(h, src2d, dst2d, za)
  out = _tc_dense(parts2, degp, h, W2l.T, W2r.T, b2l, relu=False)
  return out[:_N]


# speedup vs baseline: 1.1641x; 1.1429x over previous
import functools

import jax
import jax.numpy as jnp
from jax import lax
from jax.experimental import pallas as pl
from jax.experimental.pallas import tpu as pltpu
from jax.experimental.pallas import tpu_sc as plsc

_N = 10000
_D = 128
_NC = 2
_NS = 16
_NW = _NC * _NS
_K = 64
_NPAD = 10240
_RPT = _NPAD // _NS
_CHUNKS = 158
_EPAD = _NW * _CHUNKS * _K
_ROWS = _EPAD // _K
_ADT = jnp.bfloat16


def _sc_aggregate(with_deg):
  mesh = plsc.VectorSubcoreMesh(core_axis_name="c", subcore_axis_name="s",
                                num_cores=_NC, num_subcores=_NS)
  if with_deg:
    out_type = [jax.ShapeDtypeStruct((_NC, _NPAD, _D), _ADT),
                jax.ShapeDtypeStruct((_NW, _NPAD), jnp.float32)]
  else:
    out_type = jax.ShapeDtypeStruct((_NC, _NPAD, _D), _ADT)
  scratch = [
      pltpu.VMEM((_CHUNKS, _K), jnp.int32),
      pltpu.VMEM((_CHUNKS, _K), jnp.int32),
      pltpu.VMEM((_K, _D), _ADT),
      pltpu.VMEM((_K, _D), _ADT),
      pltpu.VMEM_SHARED((_NPAD, _D), _ADT),
      pltpu.SemaphoreType.DMA,
      pltpu.SemaphoreType.DMA,
  ]
  if with_deg:
    scratch += [pltpu.VMEM((_NPAD,), jnp.float32)]

  def body(x_hbm, src_hbm, dst_hbm, za_hbm, *rest):
    if with_deg:
      (acc_out, deg_out, src_v, dst_v, rows0, rows1, acc_sh, sem0, sem1,
       deg_v) = rest
    else:
      (acc_out, src_v, dst_v, rows0, rows1, acc_sh, sem0, sem1) = rest
    c = lax.axis_index("c")
    s = lax.axis_index("s")
    t = c * _NS + s

    pltpu.sync_copy(src_hbm.at[pl.ds(t * _CHUNKS, _CHUNKS)], src_v)
    pltpu.sync_copy(dst_hbm.at[pl.ds(t * _CHUNKS, _CHUNKS)], dst_v)
    pltpu.sync_copy(za_hbm, acc_sh.at[pl.ds(s * _RPT, _RPT)])
    if with_deg:
      zeros16 = jnp.zeros((16,), jnp.float32)
      def zero_body(i, carry):
        deg_v[pl.ds(i * 16, 16)] = zeros16
        return carry
      lax.fori_loop(0, _NPAD // 16, zero_body, 0)
    plsc.subcore_barrier()

    def g_start(j, buf, sem):
      pltpu.async_copy(x_hbm.at[src_v.at[j]], buf, sem)

    def g_wait(buf, sem):
      pltpu.make_async_copy(x_hbm.at[src_v.at[0]], buf, sem).wait()

    ones16 = jnp.ones((16,), jnp.float32)

    def scat(j, buf):
      pltpu.sync_copy(buf, acc_sh.at[dst_v.at[j]], add=True)
      if with_deg:
        for v in range(_K // 16):
          idx = dst_v[j, pl.ds(v * 16, 16)]
          plsc.addupdate_scatter(deg_v, [idx], ones16)

    def loop_body(k, carry):
      j0 = 2 * k
      g_start(j0, rows0, sem0)
      g_start(j0 + 1, rows1, sem1)
      g_wait(rows0, sem0)
      scat(j0, rows0)
      g_wait(rows1, sem1)
      scat(j0 + 1, rows1)
      return carry

    lax.fori_loop(0, _CHUNKS // 2, loop_body, 0)
    plsc.subcore_barrier()

    pltpu.sync_copy(acc_sh.at[pl.ds(s * _RPT, _RPT)],
                    acc_out.at[c, pl.ds(s * _RPT, _RPT)])
    if with_deg:
      pltpu.sync_copy(deg_v, deg_out.at[t])

  return pl.kernel(
      body, out_type=out_type, mesh=mesh, scratch_types=scratch,
      compiler_params=pltpu.CompilerParams(needs_layout_passes=False,
                                           use_tc_tiling_on_sc=False))


def _dense_body(relu, p_ref, dg_ref, x_ref, wl_ref, wr_ref, b_ref, o_ref):
  acc = (p_ref[0].astype(jnp.float32)
         + p_ref[1].astype(jnp.float32))
  deg = jnp.sum(dg_ref[...], axis=0)[:, None]
  agg = acc / jnp.maximum(deg, 1.0)
  y = (jnp.dot(agg, wl_ref[...], preferred_element_type=jnp.float32)
       + jnp.dot(x_ref[...], wr_ref[...], preferred_element_type=jnp.float32)
       + b_ref[...])
  o_ref[...] = jnp.maximum(y, 0.0) if relu else y


def _tc_dense(parts, degp, xp, wlT, wrT, b, relu):
  blk = 2048
  grid = (_NPAD // blk,)
  return pl.pallas_call(
      functools.partial(_dense_body, relu),
      grid=grid,
      in_specs=[
          pl.BlockSpec((_NC, blk, _D), lambda i: (0, i, 0)),
          pl.BlockSpec((_NW, blk), lambda i: (0, i)),
          pl.BlockSpec((blk, _D), lambda i: (i, 0)),
          pl.BlockSpec((_D, _D), lambda i: (0, 0)),
          pl.BlockSpec((_D, _D), lambda i: (0, 0)),
          pl.BlockSpec((1, _D), lambda i: (0, 0)),
      ],
      out_specs=pl.BlockSpec((blk, _D), lambda i: (i, 0)),
      out_shape=jax.ShapeDtypeStruct((_NPAD, _D), jnp.float32),
  )(parts, degp, xp, wlT, wrT, b.reshape(1, _D))


def kernel(x, edge_index, node_type, W1l, b1l, W1r, W2l, b2l, W2r):
  f32 = jnp.float32
  x = x.astype(f32)
  src = edge_index[0].astype(jnp.int32)
  dst = edge_index[1].astype(jnp.int32)
  e = src.shape[0]
  pad = _EPAD - e
  pidx = jnp.arange(pad, dtype=jnp.int32)
  psrc = pidx % _N
  pdst = _N + (pidx % (_NPAD - _N))
  src2d = jnp.concatenate([src, psrc]).reshape(_ROWS, _K)
  dst2d = jnp.concatenate([dst, pdst]).reshape(_ROWS, _K)
  za = jnp.zeros((_RPT, _D), _ADT)
  xp = jnp.concatenate([x, jnp.zeros((_NPAD - _N, _D), f32)])
  xg = xp.astype(_ADT)

  agg = _sc_aggregate(True)
  parts1, degp = agg(xg, src2d, dst2d, za)
  h = _tc_dense(parts1, degp, xp, W1l.T, W1r.T, b1l, relu=True)
  parts2, _ = agg(h.astype(_ADT), src2d, dst2d, za)
  out = _tc_dense(parts2, degp, h, W2l.T, W2r.T, b2l, relu=False)
  return out[:_N]


# speedup vs baseline: 1.3160x; 1.1304x over previous
import functools

import jax
import jax.numpy as jnp
from jax import lax
from jax.experimental import pallas as pl
from jax.experimental.pallas import tpu as pltpu
from jax.experimental.pallas import tpu_sc as plsc

_N = 10000
_D = 128
_NC = 2
_NS = 16
_NW = _NC * _NS
_K = 128
_NPAD = 10240
_RPT = _NPAD // _NS
_CHUNKS = 80
_EPAD = _NW * _CHUNKS * _K
_ROWS = _EPAD // _K
_ADT = jnp.bfloat16


def _sc_aggregate(with_deg):
  mesh = plsc.VectorSubcoreMesh(core_axis_name="c", subcore_axis_name="s",
                                num_cores=_NC, num_subcores=_NS)
  if with_deg:
    out_type = [jax.ShapeDtypeStruct((_NC, _NPAD, _D), _ADT),
                jax.ShapeDtypeStruct((_NW, _NPAD), jnp.float32)]
  else:
    out_type = jax.ShapeDtypeStruct((_NC, _NPAD, _D), _ADT)
  scratch = [
      pltpu.VMEM((_CHUNKS, _K), jnp.int32),
      pltpu.VMEM((_CHUNKS, _K), jnp.int32),
      pltpu.VMEM((_K, _D), _ADT),
      pltpu.VMEM((_K, _D), _ADT),
      pltpu.VMEM_SHARED((_NPAD, _D), _ADT),
      pltpu.SemaphoreType.DMA,
      pltpu.SemaphoreType.DMA,
  ]
  if with_deg:
    scratch += [pltpu.VMEM((_NPAD,), jnp.float32)]

  def body(x_hbm, src_hbm, dst_hbm, za_hbm, *rest):
    if with_deg:
      (acc_out, deg_out, src_v, dst_v, rows0, rows1, acc_sh, sem0, sem1,
       deg_v) = rest
    else:
      (acc_out, src_v, dst_v, rows0, rows1, acc_sh, sem0, sem1) = rest
    c = lax.axis_index("c")
    s = lax.axis_index("s")
    t = c * _NS + s

    pltpu.sync_copy(src_hbm.at[pl.ds(t * _CHUNKS, _CHUNKS)], src_v)
    pltpu.sync_copy(dst_hbm.at[pl.ds(t * _CHUNKS, _CHUNKS)], dst_v)
    pltpu.sync_copy(za_hbm, acc_sh.at[pl.ds(s * _RPT, _RPT)])
    if with_deg:
      zeros16 = jnp.zeros((16,), jnp.float32)
      def zero_body(i, carry):
        deg_v[pl.ds(i * 16, 16)] = zeros16
        return carry
      lax.fori_loop(0, _NPAD // 16, zero_body, 0)
    plsc.subcore_barrier()

    def g_start(j, buf, sem):
      pltpu.async_copy(x_hbm.at[src_v.at[j]], buf, sem)

    def g_wait(buf, sem):
      pltpu.make_async_copy(x_hbm.at[src_v.at[0]], buf, sem).wait()

    ones16 = jnp.ones((16,), jnp.float32)

    def scat(j, buf):
      pltpu.sync_copy(buf, acc_sh.at[dst_v.at[j]], add=True)
      if with_deg:
        for v in range(_K // 16):
          idx = dst_v[j, pl.ds(v * 16, 16)]
          plsc.addupdate_scatter(deg_v, [idx], ones16)

    def loop_body(k, carry):
      j0 = 2 * k
      g_start(j0, rows0, sem0)
      g_start(j0 + 1, rows1, sem1)
      g_wait(rows0, sem0)
      scat(j0, rows0)
      g_wait(rows1, sem1)
      scat(j0 + 1, rows1)
      return carry

    lax.fori_loop(0, _CHUNKS // 2, loop_body, 0)
    plsc.subcore_barrier()

    pltpu.sync_copy(acc_sh.at[pl.ds(s * _RPT, _RPT)],
                    acc_out.at[c, pl.ds(s * _RPT, _RPT)])
    if with_deg:
      pltpu.sync_copy(deg_v, deg_out.at[t])

  return pl.kernel(
      body, out_type=out_type, mesh=mesh, scratch_types=scratch,
      compiler_params=pltpu.CompilerParams(needs_layout_passes=False,
                                           use_tc_tiling_on_sc=False))


def _dense_body(relu, p_ref, dg_ref, x_ref, wl_ref, wr_ref, b_ref, o_ref):
  acc = (p_ref[0].astype(jnp.float32)
         + p_ref[1].astype(jnp.float32))
  deg = jnp.sum(dg_ref[...], axis=0)[:, None]
  agg = acc / jnp.maximum(deg, 1.0)
  y = (jnp.dot(agg, wl_ref[...], preferred_element_type=jnp.float32)
       + jnp.dot(x_ref[...], wr_ref[...], preferred_element_type=jnp.float32)
       + b_ref[...])
  o_ref[...] = jnp.maximum(y, 0.0) if relu else y


def _tc_dense(parts, degp, xp, wlT, wrT, b, relu):
  blk = 2048
  grid = (_NPAD // blk,)
  return pl.pallas_call(
      functools.partial(_dense_body, relu),
      grid=grid,
      in_specs=[
          pl.BlockSpec((_NC, blk, _D), lambda i: (0, i, 0)),
          pl.BlockSpec((_NW, blk), lambda i: (0, i)),
          pl.BlockSpec((blk, _D), lambda i: (i, 0)),
          pl.BlockSpec((_D, _D), lambda i: (0, 0)),
          pl.BlockSpec((_D, _D), lambda i: (0, 0)),
          pl.BlockSpec((1, _D), lambda i: (0, 0)),
      ],
      out_specs=pl.BlockSpec((blk, _D), lambda i: (i, 0)),
      out_shape=jax.ShapeDtypeStruct((_NPAD, _D), jnp.float32),
  )(parts, degp, xp, wlT, wrT, b.reshape(1, _D))


def kernel(x, edge_index, node_type, W1l, b1l, W1r, W2l, b2l, W2r):
  f32 = jnp.float32
  x = x.astype(f32)
  src = edge_index[0].astype(jnp.int32)
  dst = edge_index[1].astype(jnp.int32)
  e = src.shape[0]
  pad = _EPAD - e
  pidx = jnp.arange(pad, dtype=jnp.int32)
  psrc = pidx % _N
  pdst = _N + (pidx % (_NPAD - _N))
  src2d = jnp.concatenate([src, psrc]).reshape(_ROWS, _K)
  dst2d = jnp.concatenate([dst, pdst]).reshape(_ROWS, _K)
  za = jnp.zeros((_RPT, _D), _ADT)
  xp = jnp.concatenate([x, jnp.zeros((_NPAD - _N, _D), f32)])
  xg = xp.astype(_ADT)

  agg = _sc_aggregate(True)
  parts1, degp = agg(xg, src2d, dst2d, za)
  h = _tc_dense(parts1, degp, xp, W1l.T, W1r.T, b1l, relu=True)
  parts2, _ = agg(h.astype(_ADT), src2d, dst2d, za)
  out = _tc_dense(parts2, degp, h, W2l.T, W2r.T, b2l, relu=False)
  return out[:_N]


# speedup vs baseline: 1.3922x; 1.0580x over previous
import functools

import jax
import jax.numpy as jnp
from jax import lax
from jax.experimental import pallas as pl
from jax.experimental.pallas import tpu as pltpu
from jax.experimental.pallas import tpu_sc as plsc

_N = 10000
_D = 128
_NC = 2
_NS = 16
_NW = _NC * _NS
_K = 128
_NPAD = 10240
_RPT = _NPAD // _NS
_CHUNKS = 80
_EPAD = _NW * _CHUNKS * _K
_ROWS = _EPAD // _K
_ADT = jnp.bfloat16


def _sc_aggregate(with_deg):
  mesh = plsc.VectorSubcoreMesh(core_axis_name="c", subcore_axis_name="s",
                                num_cores=_NC, num_subcores=_NS)
  if with_deg:
    out_type = [jax.ShapeDtypeStruct((_NC, _NPAD, _D), _ADT),
                jax.ShapeDtypeStruct((_NW, _NPAD), jnp.float32)]
  else:
    out_type = jax.ShapeDtypeStruct((_NC, _NPAD, _D), _ADT)
  scratch = [
      pltpu.VMEM((_CHUNKS, _K), jnp.int32),
      pltpu.VMEM((_CHUNKS, _K), jnp.int32),
      pltpu.VMEM((_K, _D), _ADT),
      pltpu.VMEM((_K, _D), _ADT),
      pltpu.VMEM((_K, _D), _ADT),
      pltpu.VMEM((_K, _D), _ADT),
      pltpu.VMEM_SHARED((_NPAD, _D), _ADT),
      pltpu.SemaphoreType.DMA,
      pltpu.SemaphoreType.DMA,
      pltpu.SemaphoreType.DMA,
      pltpu.SemaphoreType.DMA,
  ]
  if with_deg:
    scratch += [pltpu.VMEM((_NPAD,), jnp.float32)]

  def body(x_hbm, src_hbm, dst_hbm, za_hbm, *rest):
    if with_deg:
      (acc_out, deg_out, src_v, dst_v, rows0, rows1, rows2, rows3, acc_sh,
       sem0, sem1, sem2, sem3, deg_v) = rest
    else:
      (acc_out, src_v, dst_v, rows0, rows1, rows2, rows3, acc_sh,
       sem0, sem1, sem2, sem3) = rest
    c = lax.axis_index("c")
    s = lax.axis_index("s")
    t = c * _NS + s

    pltpu.sync_copy(src_hbm.at[pl.ds(t * _CHUNKS, _CHUNKS)], src_v)
    pltpu.sync_copy(dst_hbm.at[pl.ds(t * _CHUNKS, _CHUNKS)], dst_v)
    pltpu.sync_copy(za_hbm, acc_sh.at[pl.ds(s * _RPT, _RPT)])
    if with_deg:
      zeros16 = jnp.zeros((16,), jnp.float32)
      def zero_body(i, carry):
        deg_v[pl.ds(i * 16, 16)] = zeros16
        return carry
      lax.fori_loop(0, _NPAD // 16, zero_body, 0)
    plsc.subcore_barrier()

    def g_start(j, buf, sem):
      pltpu.async_copy(x_hbm.at[src_v.at[j]], buf, sem)

    def g_wait(buf, sem):
      pltpu.make_async_copy(x_hbm.at[src_v.at[0]], buf, sem).wait()

    ones16 = jnp.ones((16,), jnp.float32)

    def scat(j, buf):
      pltpu.sync_copy(buf, acc_sh.at[dst_v.at[j]], add=True)
      if with_deg:
        for v in range(_K // 16):
          idx = dst_v[j, pl.ds(v * 16, 16)]
          plsc.addupdate_scatter(deg_v, [idx], ones16)

    bufs = ((rows0, sem0), (rows1, sem1), (rows2, sem2), (rows3, sem3))

    def loop_body(k, carry):
      j0 = 4 * k
      for b, (buf, sem) in enumerate(bufs):
        g_start(j0 + b, buf, sem)
      for b, (buf, sem) in enumerate(bufs):
        g_wait(buf, sem)
        scat(j0 + b, buf)
      return carry

    lax.fori_loop(0, _CHUNKS // 4, loop_body, 0)
    plsc.subcore_barrier()

    pltpu.sync_copy(acc_sh.at[pl.ds(s * _RPT, _RPT)],
                    acc_out.at[c, pl.ds(s * _RPT, _RPT)])
    if with_deg:
      pltpu.sync_copy(deg_v, deg_out.at[t])

  return pl.kernel(
      body, out_type=out_type, mesh=mesh, scratch_types=scratch,
      compiler_params=pltpu.CompilerParams(needs_layout_passes=False,
                                           use_tc_tiling_on_sc=False))


def _dense_body(relu, p_ref, dg_ref, x_ref, wl_ref, wr_ref, b_ref, o_ref):
  acc = (p_ref[0].astype(jnp.float32)
         + p_ref[1].astype(jnp.float32))
  deg = jnp.sum(dg_ref[...], axis=0)[:, None]
  agg = acc / jnp.maximum(deg, 1.0)
  y = (jnp.dot(agg, wl_ref[...], preferred_element_type=jnp.float32)
       + jnp.dot(x_ref[...], wr_ref[...], preferred_element_type=jnp.float32)
       + b_ref[...])
  o_ref[...] = jnp.maximum(y, 0.0) if relu else y


def _tc_dense(parts, degp, xp, wlT, wrT, b, relu):
  blk = 2048
  grid = (_NPAD // blk,)
  return pl.pallas_call(
      functools.partial(_dense_body, relu),
      grid=grid,
      in_specs=[
          pl.BlockSpec((_NC, blk, _D), lambda i: (0, i, 0)),
          pl.BlockSpec((_NW, blk), lambda i: (0, i)),
          pl.BlockSpec((blk, _D), lambda i: (i, 0)),
          pl.BlockSpec((_D, _D), lambda i: (0, 0)),
          pl.BlockSpec((_D, _D), lambda i: (0, 0)),
          pl.BlockSpec((1, _D), lambda i: (0, 0)),
      ],
      out_specs=pl.BlockSpec((blk, _D), lambda i: (i, 0)),
      out_shape=jax.ShapeDtypeStruct((_NPAD, _D), jnp.float32),
  )(parts, degp, xp, wlT, wrT, b.reshape(1, _D))


def kernel(x, edge_index, node_type, W1l, b1l, W1r, W2l, b2l, W2r):
  f32 = jnp.float32
  x = x.astype(f32)
  src = edge_index[0].astype(jnp.int32)
  dst = edge_index[1].astype(jnp.int32)
  e = src.shape[0]
  pad = _EPAD - e
  pidx = jnp.arange(pad, dtype=jnp.int32)
  psrc = pidx % _N
  pdst = _N + (pidx % (_NPAD - _N))
  src2d = jnp.concatenate([src, psrc]).reshape(_ROWS, _K)
  dst2d = jnp.concatenate([dst, pdst]).reshape(_ROWS, _K)
  za = jnp.zeros((_RPT, _D), _ADT)
  xp = jnp.concatenate([x, jnp.zeros((_NPAD - _N, _D), f32)])
  xg = xp.astype(_ADT)

  agg = _sc_aggregate(True)
  parts1, degp = agg(xg, src2d, dst2d, za)
  h = _tc_dense(parts1, degp, xp, W1l.T, W1r.T, b1l, relu=True)
  parts2, _ = agg(h.astype(_ADT), src2d, dst2d, za)
  out = _tc_dense(parts2, degp, h, W2l.T, W2r.T, b2l, relu=False)
  return out[:_N]


# speedup vs baseline: 1.4243x; 1.0230x over previous
import functools

import jax
import jax.numpy as jnp
from jax import lax
from jax.experimental import pallas as pl
from jax.experimental.pallas import tpu as pltpu
from jax.experimental.pallas import tpu_sc as plsc

_N = 10000
_D = 128
_NC = 2
_NS = 16
_NW = _NC * _NS
_K = 128
_NPAD = 10240
_RPT = _NPAD // _NS
_CHUNKS = 80
_EPAD = _NW * _CHUNKS * _K
_ROWS = _EPAD // _K
_ADT = jnp.bfloat16


def _sc_aggregate(with_deg):
  mesh = plsc.VectorSubcoreMesh(core_axis_name="c", subcore_axis_name="s",
                                num_cores=_NC, num_subcores=_NS)
  if with_deg:
    out_type = [jax.ShapeDtypeStruct((_NC, _NPAD, _D), _ADT),
                jax.ShapeDtypeStruct((_NW, _NPAD), jnp.float32)]
  else:
    out_type = jax.ShapeDtypeStruct((_NC, _NPAD, _D), _ADT)
  scratch = [
      pltpu.VMEM((_CHUNKS, _K), jnp.int32),
      pltpu.VMEM((_CHUNKS, _K), jnp.int32),
      pltpu.VMEM((_K, _D), _ADT),
      pltpu.VMEM((_K, _D), _ADT),
      pltpu.VMEM((_K, _D), _ADT),
      pltpu.VMEM((_K, _D), _ADT),
      pltpu.VMEM((_K, _D), _ADT),
      pltpu.VMEM_SHARED((_NPAD, _D), _ADT),
  ] + [pltpu.SemaphoreType.DMA] * 5
  if with_deg:
    scratch += [pltpu.VMEM((_NPAD,), jnp.float32)]

  def body(x_hbm, src_hbm, dst_hbm, za_hbm, *rest):
    if with_deg:
      (acc_out, deg_out, src_v, dst_v, *rb) = rest
      rows_list, (acc_sh, *sems, deg_v) = rb[:5], rb[5:]
    else:
      (acc_out, src_v, dst_v, *rb) = rest
      rows_list, (acc_sh, *sems) = rb[:5], rb[5:]
    c = lax.axis_index("c")
    s = lax.axis_index("s")
    t = c * _NS + s

    pltpu.sync_copy(src_hbm.at[pl.ds(t * _CHUNKS, _CHUNKS)], src_v)
    pltpu.sync_copy(dst_hbm.at[pl.ds(t * _CHUNKS, _CHUNKS)], dst_v)
    pltpu.sync_copy(za_hbm, acc_sh.at[pl.ds(s * _RPT, _RPT)])
    if with_deg:
      zeros16 = jnp.zeros((16,), jnp.float32)
      def zero_body(i, carry):
        deg_v[pl.ds(i * 16, 16)] = zeros16
        return carry
      lax.fori_loop(0, _NPAD // 16, zero_body, 0)
    plsc.subcore_barrier()

    def g_start(j, buf, sem):
      pltpu.async_copy(x_hbm.at[src_v.at[j]], buf, sem)

    def g_wait(buf, sem):
      pltpu.make_async_copy(x_hbm.at[src_v.at[0]], buf, sem).wait()

    ones16 = jnp.ones((16,), jnp.float32)

    def scat(j, buf):
      pltpu.sync_copy(buf, acc_sh.at[dst_v.at[j]], add=True)
      if with_deg:
        for v in range(_K // 16):
          idx = dst_v[j, pl.ds(v * 16, 16)]
          plsc.addupdate_scatter(deg_v, [idx], ones16)

    bufs = tuple(zip(rows_list, sems))

    def loop_body(k, carry):
      j0 = 5 * k
      for b, (buf, sem) in enumerate(bufs):
        g_start(j0 + b, buf, sem)
      for b, (buf, sem) in enumerate(bufs):
        g_wait(buf, sem)
        scat(j0 + b, buf)
      return carry

    lax.fori_loop(0, _CHUNKS // 5, loop_body, 0)
    plsc.subcore_barrier()

    pltpu.sync_copy(acc_sh.at[pl.ds(s * _RPT, _RPT)],
                    acc_out.at[c, pl.ds(s * _RPT, _RPT)])
    if with_deg:
      pltpu.sync_copy(deg_v, deg_out.at[t])

  return pl.kernel(
      body, out_type=out_type, mesh=mesh, scratch_types=scratch,
      compiler_params=pltpu.CompilerParams(needs_layout_passes=False,
                                           use_tc_tiling_on_sc=False))


def _dense_body(relu, p_ref, dg_ref, x_ref, wl_ref, wr_ref, b_ref, o_ref):
  acc = (p_ref[0].astype(jnp.float32)
         + p_ref[1].astype(jnp.float32))
  deg = jnp.sum(dg_ref[...], axis=0)[:, None]
  agg = acc / jnp.maximum(deg, 1.0)
  y = (jnp.dot(agg, wl_ref[...], preferred_element_type=jnp.float32)
       + jnp.dot(x_ref[...], wr_ref[...], preferred_element_type=jnp.float32)
       + b_ref[...])
  o_ref[...] = jnp.maximum(y, 0.0) if relu else y


def _tc_dense(parts, degp, xp, wlT, wrT, b, relu):
  blk = 2048
  grid = (_NPAD // blk,)
  return pl.pallas_call(
      functools.partial(_dense_body, relu),
      grid=grid,
      in_specs=[
          pl.BlockSpec((_NC, blk, _D), lambda i: (0, i, 0)),
          pl.BlockSpec((_NW, blk), lambda i: (0, i)),
          pl.BlockSpec((blk, _D), lambda i: (i, 0)),
          pl.BlockSpec((_D, _D), lambda i: (0, 0)),
          pl.BlockSpec((_D, _D), lambda i: (0, 0)),
          pl.BlockSpec((1, _D), lambda i: (0, 0)),
      ],
      out_specs=pl.BlockSpec((blk, _D), lambda i: (i, 0)),
      out_shape=jax.ShapeDtypeStruct((_NPAD, _D), jnp.float32),
  )(parts, degp, xp, wlT, wrT, b.reshape(1, _D))


def kernel(x, edge_index, node_type, W1l, b1l, W1r, W2l, b2l, W2r):
  f32 = jnp.float32
  x = x.astype(f32)
  src = edge_index[0].astype(jnp.int32)
  dst = edge_index[1].astype(jnp.int32)
  e = src.shape[0]
  pad = _EPAD - e
  pidx = jnp.arange(pad, dtype=jnp.int32)
  psrc = pidx % _N
  pdst = _N + (pidx % (_NPAD - _N))
  src2d = jnp.concatenate([src, psrc]).reshape(_ROWS, _K)
  dst2d = jnp.concatenate([dst, pdst]).reshape(_ROWS, _K)
  za = jnp.zeros((_RPT, _D), _ADT)
  xp = jnp.concatenate([x, jnp.zeros((_NPAD - _N, _D), f32)])
  xg = xp.astype(_ADT)

  agg = _sc_aggregate(True)
  parts1, degp = agg(xg, src2d, dst2d, za)
  h = _tc_dense(parts1, degp, xp, W1l.T, W1r.T, b1l, relu=True)
  parts2, _ = agg(h.astype(_ADT), src2d, dst2d, za)
  out = _tc_dense(parts2, degp, h, W2l.T, W2r.T, b2l, relu=False)
  return out[:_N]
